# baseline (speedup 1.0000x reference)
import functools

import jax
import jax.numpy as jnp
from jax import lax
from jax.experimental import pallas as pl
from jax.experimental.pallas import tpu as pltpu
from jax.experimental.pallas import tpu_sc as plsc

VOCAB = 100000
CONTEXT = 20
EMBED = 32
HIDDEN = 128
BATCH = 1024

NIDX = BATCH * CONTEXT


@functools.cache
def _gather_call(n_idx, embed):
    info = plsc.get_sparse_core_info()
    nc, ns = info.num_cores, info.num_subcores
    nw = nc * ns
    assert n_idx % nw == 0
    b_per_w = n_idx // nw
    mesh = plsc.VectorSubcoreMesh(core_axis_name="c", subcore_axis_name="s")

    @functools.partial(
        pl.kernel,
        mesh=mesh,
        out_type=jax.ShapeDtypeStruct((n_idx, embed), jnp.float32),
        scratch_types=[
            pltpu.VMEM((b_per_w,), jnp.int32),
            pltpu.VMEM((b_per_w, embed), jnp.float32),
            pltpu.SemaphoreType.DMA,
        ],
        compiler_params=pltpu.CompilerParams(use_tc_tiling_on_sc=False),
    )
    def gather_k(idx_hbm, table_hbm, out_hbm, idx_v, rows_v, sem):
        wid = lax.axis_index("s") * nc + lax.axis_index("c")
        base = wid * b_per_w
        pltpu.sync_copy(idx_hbm.at[pl.ds(base, b_per_w)], idx_v)
        pltpu.async_copy(table_hbm.at[idx_v], rows_v, sem).wait()
        pltpu.sync_copy(rows_v, out_hbm.at[pl.ds(base, b_per_w)])

    return gather_k


def _mlp_body(emb_ref, w1_ref, b1_ref, w2_ref, b2_ref, out_ref, hid_ref):
    @pl.when(pl.program_id(0) == 0)
    def _():
        h = jnp.dot(emb_ref[...], w1_ref[...],
                    preferred_element_type=jnp.float32)
        hid_ref[...] = jnp.maximum(h + b1_ref[...], 0.0)

    out_ref[...] = jnp.dot(hid_ref[...], w2_ref[...],
                           preferred_element_type=jnp.float32) + b2_ref[...]


def kernel(x, table, W1, b1, W2, b2):
    idx = x.reshape(-1).astype(jnp.int32)
    embedded = _gather_call(NIDX, EMBED)(idx, table)
    embedded = embedded.reshape(BATCH, CONTEXT * EMBED)

    vt = 1024
    nv = pl.cdiv(VOCAB, vt)
    logits = pl.pallas_call(
        _mlp_body,
        grid=(nv,),
        in_specs=[
            pl.BlockSpec((BATCH, CONTEXT * EMBED), lambda j: (0, 0)),
            pl.BlockSpec((CONTEXT * EMBED, HIDDEN), lambda j: (0, 0)),
            pl.BlockSpec((1, HIDDEN), lambda j: (0, 0)),
            pl.BlockSpec((HIDDEN, vt), lambda j: (0, j)),
            pl.BlockSpec((1, vt), lambda j: (0, j)),
        ],
        out_specs=pl.BlockSpec((BATCH, vt), lambda j: (0, j)),
        out_shape=jax.ShapeDtypeStruct((BATCH, VOCAB), jnp.float32),
        scratch_shapes=[pltpu.VMEM((BATCH, HIDDEN), jnp.float32)],
    )(embedded, W1, b1.reshape(1, HIDDEN), W2, b2.reshape(1, VOCAB))
    return logits


# speedup vs baseline: 1.0296x; 1.0296x over previous
import functools

import jax
import jax.numpy as jnp
from jax import lax
from jax.experimental import pallas as pl
from jax.experimental.pallas import tpu as pltpu
from jax.experimental.pallas import tpu_sc as plsc

VOCAB = 100000
CONTEXT = 20
EMBED = 32
HIDDEN = 128
BATCH = 1024

NIDX = BATCH * CONTEXT


@functools.cache
def _gather_call(n_idx, embed):
    info = plsc.get_sparse_core_info()
    nc, ns = info.num_cores, info.num_subcores
    nw = nc * ns
    assert n_idx % nw == 0
    b_per_w = n_idx // nw
    mesh = plsc.VectorSubcoreMesh(core_axis_name="c", subcore_axis_name="s")

    @functools.partial(
        pl.kernel,
        mesh=mesh,
        out_type=jax.ShapeDtypeStruct((n_idx, embed), jnp.float32),
        scratch_types=[
            pltpu.VMEM((b_per_w,), jnp.int32),
            pltpu.VMEM((b_per_w, embed), jnp.float32),
            pltpu.SemaphoreType.DMA,
        ],
        compiler_params=pltpu.CompilerParams(use_tc_tiling_on_sc=False),
    )
    def gather_k(idx_hbm, table_hbm, out_hbm, idx_v, rows_v, sem):
        wid = lax.axis_index("s") * nc + lax.axis_index("c")
        base = wid * b_per_w
        pltpu.sync_copy(idx_hbm.at[pl.ds(base, b_per_w)], idx_v)
        pltpu.async_copy(table_hbm.at[idx_v], rows_v, sem).wait()
        pltpu.sync_copy(rows_v, out_hbm.at[pl.ds(base, b_per_w)])

    return gather_k


def _hidden_body(emb_ref, w1_ref, b1_ref, hid_ref):
    h = jnp.dot(emb_ref[...], w1_ref[...], preferred_element_type=jnp.float32)
    hid_ref[...] = jnp.maximum(h + b1_ref[...], 0.0)


def _logits_body(hid_ref, w2_ref, b2_ref, out_ref):
    out_ref[...] = jnp.dot(hid_ref[...], w2_ref[...],
                           preferred_element_type=jnp.float32) + b2_ref[...]


def kernel(x, table, W1, b1, W2, b2):
    idx = x.reshape(-1).astype(jnp.int32)
    embedded = _gather_call(NIDX, EMBED)(idx, table)
    embedded = embedded.reshape(BATCH, CONTEXT * EMBED)

    hidden = pl.pallas_call(
        _hidden_body,
        out_shape=jax.ShapeDtypeStruct((BATCH, HIDDEN), jnp.float32),
    )(embedded, W1, b1.reshape(1, HIDDEN))

    vt = 2048
    nv = pl.cdiv(VOCAB, vt)
    logits = pl.pallas_call(
        _logits_body,
        grid=(nv,),
        in_specs=[
            pl.BlockSpec((BATCH, HIDDEN), lambda j: (0, 0)),
            pl.BlockSpec((HIDDEN, vt), lambda j: (0, j)),
            pl.BlockSpec((1, vt), lambda j: (0, j)),
        ],
        out_specs=pl.BlockSpec((BATCH, vt), lambda j: (0, j)),
        out_shape=jax.ShapeDtypeStruct((BATCH, VOCAB), jnp.float32),
        compiler_params=pltpu.CompilerParams(
            dimension_semantics=("parallel",)),
    )(hidden, W2, b2.reshape(1, VOCAB))
    return logits


# speedup vs baseline: 1.1499x; 1.1169x over previous
import functools

import jax
import jax.numpy as jnp
from jax import lax
from jax.experimental import pallas as pl
from jax.experimental.pallas import tpu as pltpu
from jax.experimental.pallas import tpu_sc as plsc

VOCAB = 100000
CONTEXT = 20
EMBED = 32
HIDDEN = 128
BATCH = 1024

NIDX = BATCH * CONTEXT

VT = 1024
ALIGNED = (VOCAB // 128) * 128
NV = (ALIGNED + VT - 1) // VT
LASTW = ALIGNED - (NV - 1) * VT
TAIL = VOCAB - ALIGNED
NBUF = 4


@functools.cache
def _gather_call(n_idx, embed):
    info = plsc.get_sparse_core_info()
    nc, ns = info.num_cores, info.num_subcores
    nw = nc * ns
    assert n_idx % nw == 0
    b_per_w = n_idx // nw
    mesh = plsc.VectorSubcoreMesh(core_axis_name="c", subcore_axis_name="s")

    @functools.partial(
        pl.kernel,
        mesh=mesh,
        out_type=jax.ShapeDtypeStruct((n_idx, embed), jnp.float32),
        scratch_types=[
            pltpu.VMEM((b_per_w,), jnp.int32),
            pltpu.VMEM((b_per_w, embed), jnp.float32),
            pltpu.SemaphoreType.DMA,
        ],
        compiler_params=pltpu.CompilerParams(use_tc_tiling_on_sc=False),
    )
    def gather_k(idx_hbm, table_hbm, out_hbm, idx_v, rows_v, sem):
        wid = lax.axis_index("s") * nc + lax.axis_index("c")
        base = wid * b_per_w
        pltpu.sync_copy(idx_hbm.at[pl.ds(base, b_per_w)], idx_v)
        pltpu.async_copy(table_hbm.at[idx_v], rows_v, sem).wait()
        pltpu.sync_copy(rows_v, out_hbm.at[pl.ds(base, b_per_w)])

    return gather_k


def _hidden_body(emb_ref, w1_ref, b1_ref, w2t_ref, b2t_ref, hid_ref, tail_ref):
    h = jnp.dot(emb_ref[...], w1_ref[...], preferred_element_type=jnp.float32)
    h = jnp.maximum(h + b1_ref[...], 0.0)
    hid_ref[...] = h
    tail_ref[...] = jnp.dot(h, w2t_ref[...],
                            preferred_element_type=jnp.float32) + b2t_ref[...]


def _logits_body(hid_ref, w2_ref, b2_ref, out_hbm, buf, sems):
    j = pl.program_id(0)
    slot = j % NBUF

    @pl.when(j >= NBUF)
    def _():
        prev = j - NBUF
        pltpu.make_async_copy(
            buf.at[slot],
            out_hbm.at[:, pl.ds(prev * VT, VT)],
            sems.at[slot],
        ).wait()

    buf[slot] = jnp.dot(hid_ref[...], w2_ref[...],
                        preferred_element_type=jnp.float32) + b2_ref[...]

    @pl.when(j < NV - 1)
    def _():
        pltpu.make_async_copy(
            buf.at[slot],
            out_hbm.at[:, pl.ds(j * VT, VT)],
            sems.at[slot],
        ).start()

    @pl.when(j == NV - 1)
    def _():
        pltpu.make_async_copy(
            buf.at[slot, :, :LASTW],
            out_hbm.at[:, pl.ds(j * VT, LASTW)],
            sems.at[slot],
        ).start()
        for k in range(NBUF):
            s = (NV - 1 - k) % NBUF
            if k == 0:
                pltpu.make_async_copy(
                    buf.at[s, :, :LASTW],
                    out_hbm.at[:, pl.ds((NV - 1) * VT, LASTW)],
                    sems.at[s],
                ).wait()
            else:
                pltpu.make_async_copy(
                    buf.at[s],
                    out_hbm.at[:, pl.ds((NV - 1 - k) * VT, VT)],
                    sems.at[s],
                ).wait()


def kernel(x, table, W1, b1, W2, b2):
    idx = x.reshape(-1).astype(jnp.int32)
    embedded = _gather_call(NIDX, EMBED)(idx, table)
    embedded = embedded.reshape(BATCH, CONTEXT * EMBED)

    hidden, tail = pl.pallas_call(
        _hidden_body,
        out_shape=[
            jax.ShapeDtypeStruct((BATCH, HIDDEN), jnp.float32),
            jax.ShapeDtypeStruct((BATCH, TAIL), jnp.float32),
        ],
    )(embedded, W1, b1.reshape(1, HIDDEN),
      W2[:, ALIGNED:], b2[ALIGNED:].reshape(1, TAIL))

    logits = pl.pallas_call(
        _logits_body,
        grid=(NV,),
        in_specs=[
            pl.BlockSpec((BATCH, HIDDEN), lambda j: (0, 0)),
            pl.BlockSpec((HIDDEN, VT), lambda j: (0, j)),
            pl.BlockSpec((1, VT), lambda j: (0, j)),
        ],
        out_specs=pl.BlockSpec(memory_space=pltpu.MemorySpace.HBM),
        out_shape=jax.ShapeDtypeStruct((BATCH, VOCAB), jnp.float32),
        scratch_shapes=[
            pltpu.VMEM((NBUF, BATCH, VT), jnp.float32),
            pltpu.SemaphoreType.DMA((NBUF,)),
        ],
    )(hidden, W2, b2.reshape(1, VOCAB))
    return lax.dynamic_update_slice(logits, tail, (0, ALIGNED))


# speedup vs baseline: 1.1514x; 1.0013x over previous
import functools

import jax
import jax.numpy as jnp
from jax import lax
from jax.experimental import pallas as pl
from jax.experimental.pallas import tpu as pltpu
from jax.experimental.pallas import tpu_sc as plsc

VOCAB = 100000
CONTEXT = 20
EMBED = 32
HIDDEN = 128
BATCH = 1024

NIDX = BATCH * CONTEXT

VT = 1024
GROUP = 4
NSTEP = 24
BULK = NSTEP * GROUP * VT
TAIL = VOCAB - BULK


@functools.cache
def _gather_call(n_idx, embed):
    info = plsc.get_sparse_core_info()
    nc, ns = info.num_cores, info.num_subcores
    nw = nc * ns
    assert n_idx % nw == 0
    b_per_w = n_idx // nw
    mesh = plsc.VectorSubcoreMesh(core_axis_name="c", subcore_axis_name="s")

    @functools.partial(
        pl.kernel,
        mesh=mesh,
        out_type=jax.ShapeDtypeStruct((n_idx, embed), jnp.float32),
        scratch_types=[
            pltpu.VMEM((b_per_w,), jnp.int32),
            pltpu.VMEM((b_per_w, embed), jnp.float32),
            pltpu.SemaphoreType.DMA,
        ],
        compiler_params=pltpu.CompilerParams(use_tc_tiling_on_sc=False),
    )
    def gather_k(idx_hbm, table_hbm, out_hbm, idx_v, rows_v, sem):
        wid = lax.axis_index("s") * nc + lax.axis_index("c")
        base = wid * b_per_w
        pltpu.sync_copy(idx_hbm.at[pl.ds(base, b_per_w)], idx_v)
        pltpu.async_copy(table_hbm.at[idx_v], rows_v, sem).wait()
        pltpu.sync_copy(rows_v, out_hbm.at[pl.ds(base, b_per_w)])

    return gather_k


def _hidden_body(emb_ref, w1_ref, b1_ref, w2t_ref, b2t_ref, hid_ref, tail_ref):
    h = jnp.dot(emb_ref[...], w1_ref[...], preferred_element_type=jnp.float32)
    h = jnp.maximum(h + b1_ref[...], 0.0)
    hid_ref[...] = h
    tail_ref[...] = jnp.dot(h, w2t_ref[...],
                            preferred_element_type=jnp.float32) + b2t_ref[...]


def _logits_body(hid_ref, w2_ref, b2_ref, out_hbm, buf, sems):
    i = pl.program_id(0)
    for k in range(GROUP):
        @pl.when(i >= 1)
        def _():
            prev = (i - 1) * GROUP + k
            pltpu.make_async_copy(
                buf.at[k],
                out_hbm.at[:, pl.ds(prev * VT, VT)],
                sems.at[k],
            ).wait()

        buf[k] = jnp.dot(
            hid_ref[...], w2_ref[:, k * VT:(k + 1) * VT],
            preferred_element_type=jnp.float32,
        ) + b2_ref[:, k * VT:(k + 1) * VT]

        pltpu.make_async_copy(
            buf.at[k],
            out_hbm.at[:, pl.ds((i * GROUP + k) * VT, VT)],
            sems.at[k],
        ).start()

    @pl.when(i == NSTEP - 1)
    def _():
        for k in range(GROUP):
            pltpu.make_async_copy(
                buf.at[k],
                out_hbm.at[:, pl.ds(((NSTEP - 1) * GROUP + k) * VT, VT)],
                sems.at[k],
            ).wait()


def kernel(x, table, W1, b1, W2, b2):
    idx = x.reshape(-1).astype(jnp.int32)
    embedded = _gather_call(NIDX, EMBED)(idx, table)
    embedded = embedded.reshape(BATCH, CONTEXT * EMBED)

    hidden, tail = pl.pallas_call(
        _hidden_body,
        out_shape=[
            jax.ShapeDtypeStruct((BATCH, HIDDEN), jnp.float32),
            jax.ShapeDtypeStruct((BATCH, TAIL), jnp.float32),
        ],
    )(embedded, W1, b1.reshape(1, HIDDEN),
      W2[:, BULK:], b2[BULK:].reshape(1, TAIL))

    logits = pl.pallas_call(
        _logits_body,
        grid=(NSTEP,),
        in_specs=[
            pl.BlockSpec((BATCH, HIDDEN), lambda i: (0, 0)),
            pl.BlockSpec((HIDDEN, GROUP * VT), lambda i: (0, i)),
            pl.BlockSpec((1, GROUP * VT), lambda i: (0, i)),
        ],
        out_specs=pl.BlockSpec(memory_space=pltpu.MemorySpace.HBM),
        out_shape=jax.ShapeDtypeStruct((BATCH, VOCAB), jnp.float32),
        scratch_shapes=[
            pltpu.VMEM((GROUP, BATCH, VT), jnp.float32),
            pltpu.SemaphoreType.DMA((GROUP,)),
        ],
    )(hidden, W2, b2.reshape(1, VOCAB))
    return lax.dynamic_update_slice(logits, tail, (0, BULK))
